# baseline (speedup 1.0000x reference)
import functools

import jax
import jax.numpy as jnp
from jax import lax
from jax.experimental import pallas as pl
from jax.experimental.pallas import tpu as pltpu
from jax.experimental.pallas import tpu_sc as plsc

_RADIUS2 = 0.2 * 0.2
_S = 32
_L = 16
_NW = 32
_CW = 128
_QA = 128
_QD = 32


def _mesh():
    return plsc.VectorSubcoreMesh(core_axis_name="c", subcore_axis_name="s")


def _wid():
    return lax.axis_index("s") * 2 + lax.axis_index("c")


def _mask_words(xr, nq_t):
    B = xr.shape[0]
    M = nq_t.shape[2]
    W = xr.shape[3]

    def body(xr_ref, nq_ref, w_ref):
        bit_pw = lax.shift_left(
            jnp.int32(1), lax.broadcasted_iota(jnp.int32, (1, 32, 1), 1)
        )
        px = xr_ref[0, 0][None]
        py = xr_ref[0, 1][None]
        pz = xr_ref[0, 2][None]
        qx = nq_ref[0, 0][:, None, None]
        qy = nq_ref[0, 1][:, None, None]
        qz = nq_ref[0, 2][:, None, None]
        dx = qx - px
        dy = qy - py
        dz = qz - pz
        d2 = dx * dx + dy * dy + dz * dz
        contrib = jnp.where(d2 < _RADIUS2, bit_pw, 0)
        w_ref[0] = jnp.sum(contrib, axis=1)

    return pl.pallas_call(
        body,
        grid=(B, M // _QA),
        in_specs=[
            pl.BlockSpec((1, 3, 32, W), lambda b, q: (b, 0, 0, 0)),
            pl.BlockSpec((1, 3, _QA), lambda b, q: (b, 0, q)),
        ],
        out_specs=pl.BlockSpec((1, _QA, W), lambda b, q: (b, q, 0)),
        out_shape=jax.ShapeDtypeStruct((B, M, W), jnp.int32),
    )(xr, nq_t)


def _extract_first32(words, B, M, N):
    W = words.shape[2]
    wpb = _NW // B
    qpw = M // wpb
    nsteps = W + _S
    blk = 8
    nblk = (nsteps + blk - 1) // blk

    @functools.partial(
        pl.kernel,
        out_type=jax.ShapeDtypeStruct((B, M, _S), jnp.int32),
        mesh=_mesh(),
        scratch_types=[
            pltpu.VMEM((qpw, W), jnp.int32),
            pltpu.SMEM((W,), jnp.int32),
            pltpu.SMEM((_S + 1,), jnp.int32),
            pltpu.VMEM((qpw, _S), jnp.int32),
        ],
    )
    def k(words_hbm, idx_hbm, wordsv, smw, smh, blkv):
        w = _wid()
        b = w // wpb
        q0 = (w % wpb) * qpw
        lanes = lax.iota(jnp.int32, _L)

        pltpu.sync_copy(words_hbm.at[b, pl.ds(q0, qpw)], wordsv)

        W1 = 48

        def per_query(jq, _):
            def stage(g0, g1):
                for g in range(g0, g1):
                    wv = wordsv[jq, pl.ds(g * _L, _L)]
                    for kk in range(_L):
                        smw[g * _L + kk] = wv[kk]

            def mk_step(Wb):
                def step(_, st):
                    widx, cnt = st
                    valid = (widx < Wb).astype(jnp.int32)
                    wslot = jnp.minimum(widx, Wb - 1)
                    wv = smw[wslot]
                    is_zero = (wv == 0).astype(jnp.int32)
                    t = wv & (-wv)
                    bits = lax.bitcast_convert_type(t.astype(jnp.float32),
                                                    jnp.int32)
                    pos = (lax.shift_right_logical(bits, 23) & 255) - 127
                    val = wslot * 32 + pos
                    hit = (1 - is_zero) * valid \
                        * (cnt < _S).astype(jnp.int32)
                    smh[jnp.minimum(cnt, _S)] = val
                    smw[wslot] = wv & (wv - 1)
                    return (widx + is_zero * valid, cnt + hit)
                return step

            def mk_block(Wb, step):
                def block(ib, st):
                    def run(st):
                        return lax.fori_loop(0, blk, step, st)
                    return lax.cond((st[1] < _S) & (st[0] < Wb), run,
                                    lambda st: st, st)
                return block

            stage(0, W1 // _L)
            nblk1 = (W1 + _S + blk - 1) // blk
            st = lax.fori_loop(
                0, nblk1, mk_block(W1, mk_step(W1)),
                (jnp.int32(0), jnp.int32(0)))

            def phase2(st):
                stage(W1 // _L, W // _L)
                nblk2 = (W - W1 + _S + blk - 1) // blk
                return lax.fori_loop(0, nblk2, mk_block(W, mk_step(W)), st)

            widx, cnt = lax.cond(st[1] < _S, phase2, lambda s: s, st)

            first = jnp.where(cnt > 0, smh[0], N - 1)
            lo = jnp.full((_L,), first, jnp.int32)
            hi = jnp.full((_L,), first, jnp.int32)
            for kk in range(_L):
                lo = jnp.where(
                    lanes == kk,
                    jnp.where(kk < cnt, smh[kk], first), lo)
                hi = jnp.where(
                    lanes == kk,
                    jnp.where(kk + _L < cnt, smh[kk + _L], first), hi)
            blkv[jq, pl.ds(0, _L)] = lo
            blkv[jq, pl.ds(_L, _L)] = hi
            return 0

        lax.fori_loop(0, qpw, per_query, 0)
        pltpu.sync_copy(blkv, idx_hbm.at[b, pl.ds(q0, qpw)])

    return k(words)


def _gather_rows(table, idx_flat, B, M):
    R = M * _S
    wpb = _NW // B
    rpw = R // wpb
    CH = 128
    nch = rpw // CH

    assert nch % 2 == 0

    @functools.partial(
        pl.kernel,
        out_type=jax.ShapeDtypeStruct((B, R, _CW), jnp.int32),
        mesh=_mesh(),
        scratch_types=[
            pltpu.VMEM((rpw,), jnp.int32),
            pltpu.VMEM((CH, _CW), jnp.int32),
            pltpu.VMEM((CH, _CW), jnp.int32),
            pltpu.SemaphoreType.DMA,
            pltpu.SemaphoreType.DMA,
        ],
    )
    def k(tab_hbm, idx_hbm, out_hbm, idxv, bufA, bufB, semA, semB):
        w = _wid()
        b = w // wpb
        r0 = (w % wpb) * rpw
        pltpu.sync_copy(idx_hbm.at[b, pl.ds(r0, rpw)], idxv)

        def gather(c, buf, sem):
            return pltpu.async_copy(
                tab_hbm.at[b].at[idxv.at[pl.ds(c * CH, CH)]], buf, sem)

        def wait(c, buf, sem):
            pltpu.make_async_copy(
                tab_hbm.at[b].at[idxv.at[pl.ds(c * CH, CH)]], buf, sem
            ).wait()

        def put(c, buf):
            pltpu.sync_copy(buf, out_hbm.at[b, pl.ds(r0 + c * CH, CH)])

        gather(0, bufA, semA)

        def pair(p, _):
            cA = 2 * p
            cB = 2 * p + 1
            gather(cB, bufB, semB)
            wait(cA, bufA, semA)
            put(cA, bufA)

            @pl.when(cA + 2 < nch)
            def _():
                gather(cA + 2, bufA, semA)

            wait(cB, bufB, semB)
            put(cB, bufB)
            return 0

        lax.fori_loop(0, nch // 2, pair, 0)

    return k(table, idx_flat)


def _transpose_out(gathered, nq_rep, B, M, C):
    R = M * _S
    RB = _QD * _S

    def body(g_ref, nx_ref, o_ref):
        w = g_ref[0]
        wt = jnp.transpose(w, (1, 0))
        lt = lax.bitcast_convert_type(
            lax.shift_left(wt, 16), jnp.float32)
        ht = lax.bitcast_convert_type(
            wt & jnp.int32(-65536), jnp.float32)
        xyz = lt[0:3, :] - nx_ref[0]
        o_ref[0] = jnp.concatenate(
            [xyz, lt[3:128, :], ht[0:C - 128, :]], axis=0)

    return pl.pallas_call(
        body,
        grid=(B, R // RB),
        in_specs=[
            pl.BlockSpec((1, RB, _CW), lambda b, r: (b, r, 0)),
            pl.BlockSpec((1, 3, RB), lambda b, r: (b, 0, r)),
        ],
        out_specs=pl.BlockSpec((1, C, RB), lambda b, r: (b, 0, r)),
        out_shape=jax.ShapeDtypeStruct((B, C, R), jnp.float32),
    )(gathered, nq_rep)


@jax.jit
def kernel(xyz, new_xyz, features):
    B, N, _ = xyz.shape
    M = new_xyz.shape[1]
    C = features.shape[1] + 3

    xyz_t = jnp.transpose(xyz, (0, 2, 1))
    nq_t = jnp.transpose(new_xyz, (0, 2, 1))
    xr = xyz_t.reshape(B, 3, N // 32, 32).transpose(0, 1, 3, 2)
    words = _mask_words(xr, nq_t)
    idx = _extract_first32(words, B, M, N)

    chans = jnp.concatenate(
        [xyz, jnp.transpose(features, (0, 2, 1))], axis=2)

    def rnd16(x):
        b = lax.bitcast_convert_type(x, jnp.uint32)
        sh16 = jnp.uint32(16)
        r = b + jnp.uint32(0x7FFF) \
            + (lax.shift_right_logical(b, sh16) & jnp.uint32(1))
        return lax.shift_right_logical(r, sh16)

    lo16 = rnd16(chans[:, :, 0:128])
    hi16 = jnp.concatenate(
        [rnd16(chans[:, :, 128:131]),
         jnp.zeros((B, N, _CW - (C - 128)), jnp.uint32)], axis=2)
    table = lax.bitcast_convert_type(
        lo16 | lax.shift_left(hi16, jnp.uint32(16)), jnp.int32)
    gathered = _gather_rows(table, idx.reshape(B, M * _S), B, M)
    nq_rep = jnp.repeat(nq_t, _S, axis=2)
    out = _transpose_out(gathered, nq_rep, B, M, C)
    return out.reshape(B, C, M, _S)


# speedup vs baseline: 1.0277x; 1.0277x over previous
import functools

import jax
import jax.numpy as jnp
from jax import lax
from jax.experimental import pallas as pl
from jax.experimental.pallas import tpu as pltpu
from jax.experimental.pallas import tpu_sc as plsc

_RADIUS2 = 0.2 * 0.2
_S = 32
_L = 16
_NW = 32
_CW = 128
_QA = 128
_QD = 32


def _mesh():
    return plsc.VectorSubcoreMesh(core_axis_name="c", subcore_axis_name="s")


def _wid():
    return lax.axis_index("s") * 2 + lax.axis_index("c")


def _mask_words(xr, nq_t):
    B = xr.shape[0]
    M = nq_t.shape[2]
    W = xr.shape[3]

    def body(xr_ref, nq_ref, w_ref):
        bit_pw = lax.shift_left(
            jnp.int32(1), lax.broadcasted_iota(jnp.int32, (1, 32, 1), 1)
        )
        px = xr_ref[0, 0][None]
        py = xr_ref[0, 1][None]
        pz = xr_ref[0, 2][None]
        qx = nq_ref[0, 0][:, None, None]
        qy = nq_ref[0, 1][:, None, None]
        qz = nq_ref[0, 2][:, None, None]
        dx = qx - px
        dy = qy - py
        dz = qz - pz
        d2 = dx * dx + dy * dy + dz * dz
        contrib = jnp.where(d2 < _RADIUS2, bit_pw, 0)
        w_ref[0] = jnp.sum(contrib, axis=1)

    return pl.pallas_call(
        body,
        grid=(B, M // _QA),
        in_specs=[
            pl.BlockSpec((1, 3, 32, W), lambda b, q: (b, 0, 0, 0)),
            pl.BlockSpec((1, 3, _QA), lambda b, q: (b, 0, q)),
        ],
        out_specs=pl.BlockSpec((1, _QA, W), lambda b, q: (b, q, 0)),
        out_shape=jax.ShapeDtypeStruct((B, M, W), jnp.int32),
    )(xr, nq_t)


def _extract_first32(words, B, M, N):
    W = words.shape[2]
    wpb = _NW // B
    qpw = M // wpb
    blk = 16

    @functools.partial(
        pl.kernel,
        out_type=jax.ShapeDtypeStruct((B, M, _S), jnp.int32),
        mesh=_mesh(),
        scratch_types=[
            pltpu.VMEM((qpw, W), jnp.int32),
            pltpu.SMEM((W,), jnp.int32),
            pltpu.SMEM((_S + 1,), jnp.int32),
            pltpu.VMEM((qpw, _S), jnp.int32),
        ],
    )
    def k(words_hbm, idx_hbm, wordsv, smw, smh, blkv):
        w = _wid()
        b = w // wpb
        q0 = (w % wpb) * qpw
        lanes = lax.iota(jnp.int32, _L)

        pltpu.sync_copy(words_hbm.at[b, pl.ds(q0, qpw)], wordsv)

        W1 = 48

        def per_query(jq, _):
            def stage(g0, g1):
                for g in range(g0, g1):
                    wv = wordsv[jq, pl.ds(g * _L, _L)]
                    for kk in range(_L):
                        smw[g * _L + kk] = wv[kk]

            def mk_step(Wb):
                def step(_, st):
                    widx, cnt, wcur = st
                    is_zero = (wcur == 0).astype(jnp.int32)
                    wadv = smw[jnp.minimum(widx + 1, Wb - 1)]
                    t = wcur & (-wcur)
                    bits = lax.bitcast_convert_type(t.astype(jnp.float32),
                                                    jnp.int32)
                    pos = (lax.shift_right_logical(bits, 23) & 255) - 127
                    val = widx * 32 + pos
                    hit = (1 - is_zero) * (cnt < _S).astype(jnp.int32)
                    smh[jnp.minimum(cnt, _S)] = val
                    wadv_ok = (widx + 1 < Wb).astype(jnp.int32)
                    wnext = jnp.where(is_zero == 1,
                                      wadv * wadv_ok, wcur & (wcur - 1))
                    return (widx + is_zero, cnt + hit, wnext)
                return step

            def mk_block(Wb, step):
                def block(ib, st):
                    def run(st):
                        return lax.fori_loop(0, blk, step, st)
                    return lax.cond(
                        (st[1] < _S) & ((st[0] < Wb - 1) | (st[2] != 0)),
                        run, lambda st: st, st)
                return block

            stage(0, W1 // _L)
            nblk1 = (W1 + _S + blk - 1) // blk
            st = lax.fori_loop(
                0, nblk1, mk_block(W1, mk_step(W1)),
                (jnp.int32(0), jnp.int32(0), smw[0]))

            def phase2(st):
                stage(W1 // _L, W // _L)
                nblk2 = (W - W1 + _S + blk - 1) // blk
                _, cnt, _wc = st
                return lax.fori_loop(0, nblk2, mk_block(W, mk_step(W)),
                                     (jnp.int32(W1), cnt, smw[W1]))

            widx, cnt, _wc = lax.cond(st[1] < _S, phase2, lambda s: s, st)

            first = jnp.where(cnt > 0, smh[0], N - 1)
            lo = jnp.full((_L,), first, jnp.int32)
            hi = jnp.full((_L,), first, jnp.int32)
            for kk in range(_L):
                lo = jnp.where(
                    lanes == kk,
                    jnp.where(kk < cnt, smh[kk], first), lo)
                hi = jnp.where(
                    lanes == kk,
                    jnp.where(kk + _L < cnt, smh[kk + _L], first), hi)
            blkv[jq, pl.ds(0, _L)] = lo
            blkv[jq, pl.ds(_L, _L)] = hi
            return 0

        lax.fori_loop(0, qpw, per_query, 0)
        pltpu.sync_copy(blkv, idx_hbm.at[b, pl.ds(q0, qpw)])

    return k(words)


def _gather_rows(table, idx_flat, B, M):
    R = M * _S
    wpb = _NW // B
    rpw = R // wpb
    CH = 128
    nch = rpw // CH

    assert nch % 2 == 0

    @functools.partial(
        pl.kernel,
        out_type=jax.ShapeDtypeStruct((B, R, _CW), jnp.int32),
        mesh=_mesh(),
        scratch_types=[
            pltpu.VMEM((rpw,), jnp.int32),
            pltpu.VMEM((CH, _CW), jnp.int32),
            pltpu.VMEM((CH, _CW), jnp.int32),
            pltpu.SemaphoreType.DMA,
            pltpu.SemaphoreType.DMA,
        ],
    )
    def k(tab_hbm, idx_hbm, out_hbm, idxv, bufA, bufB, semA, semB):
        w = _wid()
        b = w // wpb
        r0 = (w % wpb) * rpw
        pltpu.sync_copy(idx_hbm.at[b, pl.ds(r0, rpw)], idxv)

        def gather(c, buf, sem):
            return pltpu.async_copy(
                tab_hbm.at[b].at[idxv.at[pl.ds(c * CH, CH)]], buf, sem)

        def wait(c, buf, sem):
            pltpu.make_async_copy(
                tab_hbm.at[b].at[idxv.at[pl.ds(c * CH, CH)]], buf, sem
            ).wait()

        def put(c, buf):
            pltpu.sync_copy(buf, out_hbm.at[b, pl.ds(r0 + c * CH, CH)])

        gather(0, bufA, semA)

        def pair(p, _):
            cA = 2 * p
            cB = 2 * p + 1
            gather(cB, bufB, semB)
            wait(cA, bufA, semA)
            put(cA, bufA)

            @pl.when(cA + 2 < nch)
            def _():
                gather(cA + 2, bufA, semA)

            wait(cB, bufB, semB)
            put(cB, bufB)
            return 0

        lax.fori_loop(0, nch // 2, pair, 0)

    return k(table, idx_flat)


def _transpose_out(gathered, nq_rep, B, M, C):
    R = M * _S
    RB = _QD * _S

    def body(g_ref, nx_ref, o_ref):
        w = g_ref[0]
        wt = jnp.transpose(w, (1, 0))
        lt = lax.bitcast_convert_type(
            lax.shift_left(wt, 16), jnp.float32)
        ht = lax.bitcast_convert_type(
            wt & jnp.int32(-65536), jnp.float32)
        xyz = lt[0:3, :] - nx_ref[0]
        o_ref[0] = jnp.concatenate(
            [xyz, lt[3:128, :], ht[0:C - 128, :]], axis=0)

    return pl.pallas_call(
        body,
        grid=(B, R // RB),
        in_specs=[
            pl.BlockSpec((1, RB, _CW), lambda b, r: (b, r, 0)),
            pl.BlockSpec((1, 3, RB), lambda b, r: (b, 0, r)),
        ],
        out_specs=pl.BlockSpec((1, C, RB), lambda b, r: (b, 0, r)),
        out_shape=jax.ShapeDtypeStruct((B, C, R), jnp.float32),
    )(gathered, nq_rep)


@jax.jit
def kernel(xyz, new_xyz, features):
    B, N, _ = xyz.shape
    M = new_xyz.shape[1]
    C = features.shape[1] + 3

    xyz_t = jnp.transpose(xyz, (0, 2, 1))
    nq_t = jnp.transpose(new_xyz, (0, 2, 1))
    xr = xyz_t.reshape(B, 3, N // 32, 32).transpose(0, 1, 3, 2)
    words = _mask_words(xr, nq_t)
    idx = _extract_first32(words, B, M, N)

    chans = jnp.concatenate(
        [xyz, jnp.transpose(features, (0, 2, 1))], axis=2)

    def rnd16(x):
        b = lax.bitcast_convert_type(x, jnp.uint32)
        sh16 = jnp.uint32(16)
        r = b + jnp.uint32(0x7FFF) \
            + (lax.shift_right_logical(b, sh16) & jnp.uint32(1))
        return lax.shift_right_logical(r, sh16)

    lo16 = rnd16(chans[:, :, 0:128])
    hi16 = jnp.concatenate(
        [rnd16(chans[:, :, 128:131]),
         jnp.zeros((B, N, _CW - (C - 128)), jnp.uint32)], axis=2)
    table = lax.bitcast_convert_type(
        lo16 | lax.shift_left(hi16, jnp.uint32(16)), jnp.int32)
    gathered = _gather_rows(table, idx.reshape(B, M * _S), B, M)
    nq_rep = jnp.repeat(nq_t, _S, axis=2)
    out = _transpose_out(gathered, nq_rep, B, M, C)
    return out.reshape(B, C, M, _S)


# speedup vs baseline: 1.0842x; 1.0550x over previous
import functools

import jax
import jax.numpy as jnp
from jax import lax
from jax.experimental import pallas as pl
from jax.experimental.pallas import tpu as pltpu
from jax.experimental.pallas import tpu_sc as plsc

_RADIUS2 = 0.2 * 0.2
_S = 32
_L = 16
_NW = 32
_CW = 128
_QA = 128
_QD = 64


def _mesh():
    return plsc.VectorSubcoreMesh(core_axis_name="c", subcore_axis_name="s")


def _wid():
    return lax.axis_index("s") * 2 + lax.axis_index("c")


def _mask_words(xr, nq_t):
    B = xr.shape[0]
    M = nq_t.shape[2]
    W = xr.shape[3]

    def body(xr_ref, nq_ref, w_ref):
        bit_pw = lax.shift_left(
            jnp.int32(1), lax.broadcasted_iota(jnp.int32, (1, 32, 1), 1)
        )
        px = xr_ref[0, 0][None]
        py = xr_ref[0, 1][None]
        pz = xr_ref[0, 2][None]
        qx = nq_ref[0, 0][:, None, None]
        qy = nq_ref[0, 1][:, None, None]
        qz = nq_ref[0, 2][:, None, None]
        dx = qx - px
        dy = qy - py
        dz = qz - pz
        d2 = dx * dx + dy * dy + dz * dz
        contrib = jnp.where(d2 < _RADIUS2, bit_pw, 0)
        w_ref[0] = jnp.sum(contrib, axis=1)

    return pl.pallas_call(
        body,
        grid=(B, M // _QA),
        in_specs=[
            pl.BlockSpec((1, 3, 32, W), lambda b, q: (b, 0, 0, 0)),
            pl.BlockSpec((1, 3, _QA), lambda b, q: (b, 0, q)),
        ],
        out_specs=pl.BlockSpec((1, _QA, W), lambda b, q: (b, q, 0)),
        out_shape=jax.ShapeDtypeStruct((B, M, W), jnp.int32),
    )(xr, nq_t)


def _extract_first32(words, B, M, N):
    W = words.shape[2]
    wpb = _NW // B
    qpw = M // wpb
    blk = 16

    @functools.partial(
        pl.kernel,
        out_type=jax.ShapeDtypeStruct((B, M, _S), jnp.int32),
        mesh=_mesh(),
        scratch_types=[
            pltpu.VMEM((qpw, W), jnp.int32),
            pltpu.SMEM((W,), jnp.int32),
            pltpu.SMEM((_S + 1,), jnp.int32),
            pltpu.VMEM((qpw, _S), jnp.int32),
        ],
    )
    def k(words_hbm, idx_hbm, wordsv, smw, smh, blkv):
        w = _wid()
        b = w // wpb
        q0 = (w % wpb) * qpw
        lanes = lax.iota(jnp.int32, _L)

        pltpu.sync_copy(words_hbm.at[b, pl.ds(q0, qpw)], wordsv)

        W1 = 48

        def per_query(jq, _):
            def stage(g0, g1):
                for g in range(g0, g1):
                    wv = wordsv[jq, pl.ds(g * _L, _L)]
                    for kk in range(_L):
                        smw[g * _L + kk] = wv[kk]

            def mk_step(Wb):
                def step(_, st):
                    widx, cnt, wcur = st
                    is_zero = (wcur == 0).astype(jnp.int32)
                    wadv = smw[jnp.minimum(widx + 1, Wb - 1)]
                    t = wcur & (-wcur)
                    bits = lax.bitcast_convert_type(t.astype(jnp.float32),
                                                    jnp.int32)
                    pos = (lax.shift_right_logical(bits, 23) & 255) - 127
                    val = widx * 32 + pos
                    hit = (1 - is_zero) * (cnt < _S).astype(jnp.int32)
                    smh[jnp.minimum(cnt, _S)] = val
                    wadv_ok = (widx + 1 < Wb).astype(jnp.int32)
                    wnext = jnp.where(is_zero == 1,
                                      wadv * wadv_ok, wcur & (wcur - 1))
                    return (widx + is_zero, cnt + hit, wnext)
                return step

            def mk_block(Wb, step):
                def block(ib, st):
                    def run(st):
                        return lax.fori_loop(0, blk, step, st)
                    return lax.cond(
                        (st[1] < _S) & ((st[0] < Wb - 1) | (st[2] != 0)),
                        run, lambda st: st, st)
                return block

            stage(0, W1 // _L)
            nblk1 = (W1 + _S + blk - 1) // blk
            st = lax.fori_loop(
                0, nblk1, mk_block(W1, mk_step(W1)),
                (jnp.int32(0), jnp.int32(0), smw[0]))

            def phase2(st):
                stage(W1 // _L, W // _L)
                nblk2 = (W - W1 + _S + blk - 1) // blk
                _, cnt, _wc = st
                return lax.fori_loop(0, nblk2, mk_block(W, mk_step(W)),
                                     (jnp.int32(W1), cnt, smw[W1]))

            widx, cnt, _wc = lax.cond(st[1] < _S, phase2, lambda s: s, st)

            first = jnp.where(cnt > 0, smh[0], N - 1)
            lo = jnp.full((_L,), first, jnp.int32)
            hi = jnp.full((_L,), first, jnp.int32)
            for kk in range(_L):
                lo = jnp.where(
                    lanes == kk,
                    jnp.where(kk < cnt, smh[kk], first), lo)
                hi = jnp.where(
                    lanes == kk,
                    jnp.where(kk + _L < cnt, smh[kk + _L], first), hi)
            blkv[jq, pl.ds(0, _L)] = lo
            blkv[jq, pl.ds(_L, _L)] = hi
            return 0

        lax.fori_loop(0, qpw, per_query, 0)
        pltpu.sync_copy(blkv, idx_hbm.at[b, pl.ds(q0, qpw)])

    return k(words)


def _gather_rows(table, idx_flat, B, M):
    R = M * _S
    wpb = _NW // B
    rpw = R // wpb
    CH = 128
    nch = rpw // CH

    assert nch % 2 == 0

    @functools.partial(
        pl.kernel,
        out_type=jax.ShapeDtypeStruct((B, R, _CW), jnp.int32),
        mesh=_mesh(),
        scratch_types=[
            pltpu.VMEM((rpw,), jnp.int32),
            pltpu.VMEM((CH, _CW), jnp.int32),
            pltpu.VMEM((CH, _CW), jnp.int32),
            pltpu.SemaphoreType.DMA,
            pltpu.SemaphoreType.DMA,
        ],
    )
    def k(tab_hbm, idx_hbm, out_hbm, idxv, bufA, bufB, semA, semB):
        w = _wid()
        b = w // wpb
        r0 = (w % wpb) * rpw
        pltpu.sync_copy(idx_hbm.at[b, pl.ds(r0, rpw)], idxv)

        def gather(c, buf, sem):
            return pltpu.async_copy(
                tab_hbm.at[b].at[idxv.at[pl.ds(c * CH, CH)]], buf, sem)

        def wait(c, buf, sem):
            pltpu.make_async_copy(
                tab_hbm.at[b].at[idxv.at[pl.ds(c * CH, CH)]], buf, sem
            ).wait()

        def put(c, buf):
            pltpu.sync_copy(buf, out_hbm.at[b, pl.ds(r0 + c * CH, CH)])

        gather(0, bufA, semA)

        def pair(p, _):
            cA = 2 * p
            cB = 2 * p + 1
            gather(cB, bufB, semB)
            wait(cA, bufA, semA)
            put(cA, bufA)

            @pl.when(cA + 2 < nch)
            def _():
                gather(cA + 2, bufA, semA)

            wait(cB, bufB, semB)
            put(cB, bufB)
            return 0

        lax.fori_loop(0, nch // 2, pair, 0)

    return k(table, idx_flat)


def _transpose_out(gathered, nq_rep, B, M, C):
    R = M * _S
    RB = _QD * _S

    def body(g_ref, nx_ref, o_ref):
        w = g_ref[0]
        wt = jnp.transpose(w, (1, 0))
        lt = lax.bitcast_convert_type(
            lax.shift_left(wt, 16), jnp.float32)
        ht = lax.bitcast_convert_type(
            wt & jnp.int32(-65536), jnp.float32)
        xyz = lt[0:3, :] - nx_ref[0]
        o_ref[0] = jnp.concatenate(
            [xyz, lt[3:128, :], ht[0:C - 128, :]], axis=0)

    return pl.pallas_call(
        body,
        grid=(B, R // RB),
        in_specs=[
            pl.BlockSpec((1, RB, _CW), lambda b, r: (b, r, 0)),
            pl.BlockSpec((1, 3, RB), lambda b, r: (b, 0, r)),
        ],
        out_specs=pl.BlockSpec((1, C, RB), lambda b, r: (b, 0, r)),
        out_shape=jax.ShapeDtypeStruct((B, C, R), jnp.float32),
    )(gathered, nq_rep)


@jax.jit
def kernel(xyz, new_xyz, features):
    B, N, _ = xyz.shape
    M = new_xyz.shape[1]
    C = features.shape[1] + 3

    xyz_t = jnp.transpose(xyz, (0, 2, 1))
    nq_t = jnp.transpose(new_xyz, (0, 2, 1))
    xr = xyz_t.reshape(B, 3, N // 32, 32).transpose(0, 1, 3, 2)
    words = _mask_words(xr, nq_t)
    idx = _extract_first32(words, B, M, N)

    chans = jnp.concatenate(
        [xyz, jnp.transpose(features, (0, 2, 1))], axis=2)

    def rnd16(x):
        b = lax.bitcast_convert_type(x, jnp.uint32)
        sh16 = jnp.uint32(16)
        r = b + jnp.uint32(0x7FFF) \
            + (lax.shift_right_logical(b, sh16) & jnp.uint32(1))
        return lax.shift_right_logical(r, sh16)

    lo16 = rnd16(chans[:, :, 0:128])
    hi16 = jnp.concatenate(
        [rnd16(chans[:, :, 128:131]),
         jnp.zeros((B, N, _CW - (C - 128)), jnp.uint32)], axis=2)
    table = lax.bitcast_convert_type(
        lo16 | lax.shift_left(hi16, jnp.uint32(16)), jnp.int32)
    gathered = _gather_rows(table, idx.reshape(B, M * _S), B, M)
    nq_rep = jnp.repeat(nq_t, _S, axis=2)
    out = _transpose_out(gathered, nq_rep, B, M, C)
    return out.reshape(B, C, M, _S)


# speedup vs baseline: 1.1308x; 1.0430x over previous
import functools

import jax
import jax.numpy as jnp
from jax import lax
from jax.experimental import pallas as pl
from jax.experimental.pallas import tpu as pltpu
from jax.experimental.pallas import tpu_sc as plsc

_RADIUS2 = 0.2 * 0.2
_S = 32
_L = 16
_NW = 32
_CW = 128
_QA = 128
_QD = 128


def _mesh():
    return plsc.VectorSubcoreMesh(core_axis_name="c", subcore_axis_name="s")


def _wid():
    return lax.axis_index("s") * 2 + lax.axis_index("c")


def _mask_words(xr, nq_t):
    B = xr.shape[0]
    M = nq_t.shape[2]
    W = xr.shape[3]

    def body(xr_ref, nq_ref, w_ref):
        bit_pw = lax.shift_left(
            jnp.int32(1), lax.broadcasted_iota(jnp.int32, (1, 32, 1), 1)
        )
        px = xr_ref[0, 0][None]
        py = xr_ref[0, 1][None]
        pz = xr_ref[0, 2][None]
        qx = nq_ref[0, 0][:, None, None]
        qy = nq_ref[0, 1][:, None, None]
        qz = nq_ref[0, 2][:, None, None]
        dx = qx - px
        dy = qy - py
        dz = qz - pz
        d2 = dx * dx + dy * dy + dz * dz
        contrib = jnp.where(d2 < _RADIUS2, bit_pw, 0)
        w_ref[0] = jnp.sum(contrib, axis=1)

    return pl.pallas_call(
        body,
        grid=(B, M // _QA),
        in_specs=[
            pl.BlockSpec((1, 3, 32, W), lambda b, q: (b, 0, 0, 0)),
            pl.BlockSpec((1, 3, _QA), lambda b, q: (b, 0, q)),
        ],
        out_specs=pl.BlockSpec((1, _QA, W), lambda b, q: (b, q, 0)),
        out_shape=jax.ShapeDtypeStruct((B, M, W), jnp.int32),
    )(xr, nq_t)


def _extract_first32(words, B, M, N):
    W = words.shape[2]
    wpb = _NW // B
    qpw = M // wpb
    blk = 16

    @functools.partial(
        pl.kernel,
        out_type=jax.ShapeDtypeStruct((B, M, _S), jnp.int32),
        mesh=_mesh(),
        scratch_types=[
            pltpu.VMEM((qpw, W), jnp.int32),
            pltpu.SMEM((W,), jnp.int32),
            pltpu.SMEM((_S + 1,), jnp.int32),
            pltpu.VMEM((qpw, _S), jnp.int32),
        ],
    )
    def k(words_hbm, idx_hbm, wordsv, smw, smh, blkv):
        w = _wid()
        b = w // wpb
        q0 = (w % wpb) * qpw
        lanes = lax.iota(jnp.int32, _L)

        pltpu.sync_copy(words_hbm.at[b, pl.ds(q0, qpw)], wordsv)

        W1 = 48

        def per_query(jq, _):
            def stage(g0, g1):
                for g in range(g0, g1):
                    wv = wordsv[jq, pl.ds(g * _L, _L)]
                    for kk in range(_L):
                        smw[g * _L + kk] = wv[kk]

            def mk_step(Wb):
                def step(_, st):
                    widx, cnt, wcur = st
                    is_zero = (wcur == 0).astype(jnp.int32)
                    wadv = smw[jnp.minimum(widx + 1, Wb - 1)]
                    t = wcur & (-wcur)
                    bits = lax.bitcast_convert_type(t.astype(jnp.float32),
                                                    jnp.int32)
                    pos = (lax.shift_right_logical(bits, 23) & 255) - 127
                    val = widx * 32 + pos
                    hit = (1 - is_zero) * (cnt < _S).astype(jnp.int32)
                    smh[jnp.minimum(cnt, _S)] = val
                    wadv_ok = (widx + 1 < Wb).astype(jnp.int32)
                    wnext = jnp.where(is_zero == 1,
                                      wadv * wadv_ok, wcur & (wcur - 1))
                    return (widx + is_zero, cnt + hit, wnext)
                return step

            def mk_block(Wb, step):
                def block(ib, st):
                    def run(st):
                        return lax.fori_loop(0, blk, step, st)
                    return lax.cond(
                        (st[1] < _S) & ((st[0] < Wb - 1) | (st[2] != 0)),
                        run, lambda st: st, st)
                return block

            stage(0, W1 // _L)
            nblk1 = (W1 + _S + blk - 1) // blk
            st = lax.fori_loop(
                0, nblk1, mk_block(W1, mk_step(W1)),
                (jnp.int32(0), jnp.int32(0), smw[0]))

            def phase2(st):
                stage(W1 // _L, W // _L)
                nblk2 = (W - W1 + _S + blk - 1) // blk
                _, cnt, _wc = st
                return lax.fori_loop(0, nblk2, mk_block(W, mk_step(W)),
                                     (jnp.int32(W1), cnt, smw[W1]))

            widx, cnt, _wc = lax.cond(st[1] < _S, phase2, lambda s: s, st)

            first = jnp.where(cnt > 0, smh[0], N - 1)
            lo = jnp.full((_L,), first, jnp.int32)
            hi = jnp.full((_L,), first, jnp.int32)
            for kk in range(_L):
                lo = jnp.where(
                    lanes == kk,
                    jnp.where(kk < cnt, smh[kk], first), lo)
                hi = jnp.where(
                    lanes == kk,
                    jnp.where(kk + _L < cnt, smh[kk + _L], first), hi)
            blkv[jq, pl.ds(0, _L)] = lo
            blkv[jq, pl.ds(_L, _L)] = hi
            return 0

        lax.fori_loop(0, qpw, per_query, 0)
        pltpu.sync_copy(blkv, idx_hbm.at[b, pl.ds(q0, qpw)])

    return k(words)


def _gather_rows(table, idx_flat, B, M):
    R = M * _S
    wpb = _NW // B
    rpw = R // wpb
    CH = 128
    nch = rpw // CH

    assert nch % 2 == 0

    @functools.partial(
        pl.kernel,
        out_type=jax.ShapeDtypeStruct((B, R, _CW), jnp.int32),
        mesh=_mesh(),
        scratch_types=[
            pltpu.VMEM((rpw,), jnp.int32),
            pltpu.VMEM((CH, _CW), jnp.int32),
            pltpu.VMEM((CH, _CW), jnp.int32),
            pltpu.SemaphoreType.DMA,
            pltpu.SemaphoreType.DMA,
        ],
    )
    def k(tab_hbm, idx_hbm, out_hbm, idxv, bufA, bufB, semA, semB):
        w = _wid()
        b = w // wpb
        r0 = (w % wpb) * rpw
        pltpu.sync_copy(idx_hbm.at[b, pl.ds(r0, rpw)], idxv)

        def gather(c, buf, sem):
            return pltpu.async_copy(
                tab_hbm.at[b].at[idxv.at[pl.ds(c * CH, CH)]], buf, sem)

        def wait(c, buf, sem):
            pltpu.make_async_copy(
                tab_hbm.at[b].at[idxv.at[pl.ds(c * CH, CH)]], buf, sem
            ).wait()

        def put(c, buf):
            pltpu.sync_copy(buf, out_hbm.at[b, pl.ds(r0 + c * CH, CH)])

        gather(0, bufA, semA)

        def pair(p, _):
            cA = 2 * p
            cB = 2 * p + 1
            gather(cB, bufB, semB)
            wait(cA, bufA, semA)
            put(cA, bufA)

            @pl.when(cA + 2 < nch)
            def _():
                gather(cA + 2, bufA, semA)

            wait(cB, bufB, semB)
            put(cB, bufB)
            return 0

        lax.fori_loop(0, nch // 2, pair, 0)

    return k(table, idx_flat)


def _transpose_out(gathered, nq_rep, B, M, C):
    R = M * _S
    RB = _QD * _S

    def body(g_ref, nx_ref, o_ref):
        w = g_ref[0]
        wt = jnp.transpose(w, (1, 0))
        lt = lax.bitcast_convert_type(
            lax.shift_left(wt, 16), jnp.float32)
        ht = lax.bitcast_convert_type(
            wt & jnp.int32(-65536), jnp.float32)
        xyz = lt[0:3, :] - nx_ref[0]
        o_ref[0] = jnp.concatenate(
            [xyz, lt[3:128, :], ht[0:C - 128, :]], axis=0)

    return pl.pallas_call(
        body,
        grid=(B, R // RB),
        in_specs=[
            pl.BlockSpec((1, RB, _CW), lambda b, r: (b, r, 0)),
            pl.BlockSpec((1, 3, RB), lambda b, r: (b, 0, r)),
        ],
        out_specs=pl.BlockSpec((1, C, RB), lambda b, r: (b, 0, r)),
        out_shape=jax.ShapeDtypeStruct((B, C, R), jnp.float32),
    )(gathered, nq_rep)


@jax.jit
def kernel(xyz, new_xyz, features):
    B, N, _ = xyz.shape
    M = new_xyz.shape[1]
    C = features.shape[1] + 3

    xyz_t = jnp.transpose(xyz, (0, 2, 1))
    nq_t = jnp.transpose(new_xyz, (0, 2, 1))
    xr = xyz_t.reshape(B, 3, N // 32, 32).transpose(0, 1, 3, 2)
    words = _mask_words(xr, nq_t)
    idx = _extract_first32(words, B, M, N)

    chans = jnp.concatenate(
        [xyz, jnp.transpose(features, (0, 2, 1))], axis=2)

    def rnd16(x):
        b = lax.bitcast_convert_type(x, jnp.uint32)
        sh16 = jnp.uint32(16)
        r = b + jnp.uint32(0x7FFF) \
            + (lax.shift_right_logical(b, sh16) & jnp.uint32(1))
        return lax.shift_right_logical(r, sh16)

    lo16 = rnd16(chans[:, :, 0:128])
    hi16 = jnp.concatenate(
        [rnd16(chans[:, :, 128:131]),
         jnp.zeros((B, N, _CW - (C - 128)), jnp.uint32)], axis=2)
    table = lax.bitcast_convert_type(
        lo16 | lax.shift_left(hi16, jnp.uint32(16)), jnp.int32)
    gathered = _gather_rows(table, idx.reshape(B, M * _S), B, M)
    nq_rep = jnp.repeat(nq_t, _S, axis=2)
    out = _transpose_out(gathered, nq_rep, B, M, C)
    return out.reshape(B, C, M, _S)


# speedup vs baseline: 1.1481x; 1.0152x over previous
import functools

import jax
import jax.numpy as jnp
from jax import lax
from jax.experimental import pallas as pl
from jax.experimental.pallas import tpu as pltpu
from jax.experimental.pallas import tpu_sc as plsc

_RADIUS2 = 0.2 * 0.2
_S = 32
_L = 16
_NW = 32
_CW = 128
_QA = 128
_QD = 256


def _mesh():
    return plsc.VectorSubcoreMesh(core_axis_name="c", subcore_axis_name="s")


def _wid():
    return lax.axis_index("s") * 2 + lax.axis_index("c")


def _mask_words(xr, nq_t):
    B = xr.shape[0]
    M = nq_t.shape[2]
    W = xr.shape[3]

    def body(xr_ref, nq_ref, w_ref):
        bit_pw = lax.shift_left(
            jnp.int32(1), lax.broadcasted_iota(jnp.int32, (1, 32, 1), 1)
        )
        px = xr_ref[0, 0][None]
        py = xr_ref[0, 1][None]
        pz = xr_ref[0, 2][None]
        qx = nq_ref[0, 0][:, None, None]
        qy = nq_ref[0, 1][:, None, None]
        qz = nq_ref[0, 2][:, None, None]
        dx = qx - px
        dy = qy - py
        dz = qz - pz
        d2 = dx * dx + dy * dy + dz * dz
        contrib = jnp.where(d2 < _RADIUS2, bit_pw, 0)
        w_ref[0] = jnp.sum(contrib, axis=1)

    return pl.pallas_call(
        body,
        grid=(B, M // _QA),
        in_specs=[
            pl.BlockSpec((1, 3, 32, W), lambda b, q: (b, 0, 0, 0)),
            pl.BlockSpec((1, 3, _QA), lambda b, q: (b, 0, q)),
        ],
        out_specs=pl.BlockSpec((1, _QA, W), lambda b, q: (b, q, 0)),
        out_shape=jax.ShapeDtypeStruct((B, M, W), jnp.int32),
    )(xr, nq_t)


def _extract_first32(words, B, M, N):
    W = words.shape[2]
    wpb = _NW // B
    qpw = M // wpb
    blk = 16

    @functools.partial(
        pl.kernel,
        out_type=jax.ShapeDtypeStruct((B, M, _S), jnp.int32),
        mesh=_mesh(),
        scratch_types=[
            pltpu.VMEM((qpw, W), jnp.int32),
            pltpu.SMEM((W,), jnp.int32),
            pltpu.SMEM((_S + 1,), jnp.int32),
            pltpu.VMEM((qpw, _S), jnp.int32),
        ],
    )
    def k(words_hbm, idx_hbm, wordsv, smw, smh, blkv):
        w = _wid()
        b = w // wpb
        q0 = (w % wpb) * qpw
        lanes = lax.iota(jnp.int32, _L)

        pltpu.sync_copy(words_hbm.at[b, pl.ds(q0, qpw)], wordsv)

        W1 = 48

        def per_query(jq, _):
            def stage(g0, g1):
                for g in range(g0, g1):
                    wv = wordsv[jq, pl.ds(g * _L, _L)]
                    for kk in range(_L):
                        smw[g * _L + kk] = wv[kk]

            def mk_step(Wb):
                def step(_, st):
                    widx, cnt, wcur = st
                    is_zero = (wcur == 0).astype(jnp.int32)
                    wadv = smw[jnp.minimum(widx + 1, Wb - 1)]
                    t = wcur & (-wcur)
                    bits = lax.bitcast_convert_type(t.astype(jnp.float32),
                                                    jnp.int32)
                    pos = (lax.shift_right_logical(bits, 23) & 255) - 127
                    val = widx * 32 + pos
                    hit = (1 - is_zero) * (cnt < _S).astype(jnp.int32)
                    smh[jnp.minimum(cnt, _S)] = val
                    wadv_ok = (widx + 1 < Wb).astype(jnp.int32)
                    wnext = jnp.where(is_zero == 1,
                                      wadv * wadv_ok, wcur & (wcur - 1))
                    return (widx + is_zero, cnt + hit, wnext)
                return step

            def mk_block(Wb, step):
                def block(ib, st):
                    def run(st):
                        return lax.fori_loop(0, blk, step, st)
                    return lax.cond(
                        (st[1] < _S) & ((st[0] < Wb - 1) | (st[2] != 0)),
                        run, lambda st: st, st)
                return block

            stage(0, W1 // _L)
            nblk1 = (W1 + _S + blk - 1) // blk
            st = lax.fori_loop(
                0, nblk1, mk_block(W1, mk_step(W1)),
                (jnp.int32(0), jnp.int32(0), smw[0]))

            def phase2(st):
                stage(W1 // _L, W // _L)
                nblk2 = (W - W1 + _S + blk - 1) // blk
                _, cnt, _wc = st
                return lax.fori_loop(0, nblk2, mk_block(W, mk_step(W)),
                                     (jnp.int32(W1), cnt, smw[W1]))

            widx, cnt, _wc = lax.cond(st[1] < _S, phase2, lambda s: s, st)

            first = jnp.where(cnt > 0, smh[0], N - 1)
            lo = jnp.full((_L,), first, jnp.int32)
            hi = jnp.full((_L,), first, jnp.int32)
            for kk in range(_L):
                lo = jnp.where(
                    lanes == kk,
                    jnp.where(kk < cnt, smh[kk], first), lo)
                hi = jnp.where(
                    lanes == kk,
                    jnp.where(kk + _L < cnt, smh[kk + _L], first), hi)
            blkv[jq, pl.ds(0, _L)] = lo
            blkv[jq, pl.ds(_L, _L)] = hi
            return 0

        lax.fori_loop(0, qpw, per_query, 0)
        pltpu.sync_copy(blkv, idx_hbm.at[b, pl.ds(q0, qpw)])

    return k(words)


def _gather_rows(table, idx_flat, B, M):
    R = M * _S
    wpb = _NW // B
    rpw = R // wpb
    CH = 128
    nch = rpw // CH

    assert nch % 2 == 0

    @functools.partial(
        pl.kernel,
        out_type=jax.ShapeDtypeStruct((B, R, _CW), jnp.int32),
        mesh=_mesh(),
        scratch_types=[
            pltpu.VMEM((rpw,), jnp.int32),
            pltpu.VMEM((CH, _CW), jnp.int32),
            pltpu.VMEM((CH, _CW), jnp.int32),
            pltpu.SemaphoreType.DMA,
            pltpu.SemaphoreType.DMA,
        ],
    )
    def k(tab_hbm, idx_hbm, out_hbm, idxv, bufA, bufB, semA, semB):
        w = _wid()
        b = w // wpb
        r0 = (w % wpb) * rpw
        pltpu.sync_copy(idx_hbm.at[b, pl.ds(r0, rpw)], idxv)

        def gather(c, buf, sem):
            return pltpu.async_copy(
                tab_hbm.at[b].at[idxv.at[pl.ds(c * CH, CH)]], buf, sem)

        def wait(c, buf, sem):
            pltpu.make_async_copy(
                tab_hbm.at[b].at[idxv.at[pl.ds(c * CH, CH)]], buf, sem
            ).wait()

        def put(c, buf):
            pltpu.sync_copy(buf, out_hbm.at[b, pl.ds(r0 + c * CH, CH)])

        gather(0, bufA, semA)

        def pair(p, _):
            cA = 2 * p
            cB = 2 * p + 1
            gather(cB, bufB, semB)
            wait(cA, bufA, semA)
            put(cA, bufA)

            @pl.when(cA + 2 < nch)
            def _():
                gather(cA + 2, bufA, semA)

            wait(cB, bufB, semB)
            put(cB, bufB)
            return 0

        lax.fori_loop(0, nch // 2, pair, 0)

    return k(table, idx_flat)


def _transpose_out(gathered, nq_rep, B, M, C):
    R = M * _S
    RB = _QD * _S

    def body(g_ref, nx_ref, o_ref):
        w = g_ref[0]
        wt = jnp.transpose(w, (1, 0))
        lt = lax.bitcast_convert_type(
            lax.shift_left(wt, 16), jnp.float32)
        ht = lax.bitcast_convert_type(
            wt & jnp.int32(-65536), jnp.float32)
        xyz = lt[0:3, :] - nx_ref[0]
        o_ref[0] = jnp.concatenate(
            [xyz, lt[3:128, :], ht[0:C - 128, :]], axis=0)

    return pl.pallas_call(
        body,
        grid=(B, R // RB),
        in_specs=[
            pl.BlockSpec((1, RB, _CW), lambda b, r: (b, r, 0)),
            pl.BlockSpec((1, 3, RB), lambda b, r: (b, 0, r)),
        ],
        out_specs=pl.BlockSpec((1, C, RB), lambda b, r: (b, 0, r)),
        out_shape=jax.ShapeDtypeStruct((B, C, R), jnp.float32),
    )(gathered, nq_rep)


@jax.jit
def kernel(xyz, new_xyz, features):
    B, N, _ = xyz.shape
    M = new_xyz.shape[1]
    C = features.shape[1] + 3

    xyz_t = jnp.transpose(xyz, (0, 2, 1))
    nq_t = jnp.transpose(new_xyz, (0, 2, 1))
    xr = xyz_t.reshape(B, 3, N // 32, 32).transpose(0, 1, 3, 2)
    words = _mask_words(xr, nq_t)
    idx = _extract_first32(words, B, M, N)

    chans = jnp.concatenate(
        [xyz, jnp.transpose(features, (0, 2, 1))], axis=2)

    def rnd16(x):
        b = lax.bitcast_convert_type(x, jnp.uint32)
        sh16 = jnp.uint32(16)
        r = b + jnp.uint32(0x7FFF) \
            + (lax.shift_right_logical(b, sh16) & jnp.uint32(1))
        return lax.shift_right_logical(r, sh16)

    lo16 = rnd16(chans[:, :, 0:128])
    hi16 = jnp.concatenate(
        [rnd16(chans[:, :, 128:131]),
         jnp.zeros((B, N, _CW - (C - 128)), jnp.uint32)], axis=2)
    table = lax.bitcast_convert_type(
        lo16 | lax.shift_left(hi16, jnp.uint32(16)), jnp.int32)
    gathered = _gather_rows(table, idx.reshape(B, M * _S), B, M)
    nq_rep = jnp.repeat(nq_t, _S, axis=2)
    out = _transpose_out(gathered, nq_rep, B, M, C)
    return out.reshape(B, C, M, _S)
